# baseline (speedup 1.0000x reference)
import functools

import jax
import jax.numpy as jnp
from jax import lax
from jax.experimental import pallas as pl
from jax.experimental.pallas import tpu as pltpu
from jax.experimental.pallas import tpu_sc as plsc

_B, _T, _D = 16, 1024, 256
_K = 1024
_N = _B * _T
_BLK = 512
_COMMIT = 0.25

_NSPLIT = 4
_NSUB = _N // _NSPLIT
_NBLK = _NSUB // _BLK

_NW = 32
_BPW = _NSUB // _NW
_CH = 128
_NCH = _BPW // _CH


def _dist_argmin_kernel(x_ref, cb_ref, a2_ref, b2_ref, idx_ref, bsum_ref):
    i = pl.program_id(0)
    x = x_ref[...]
    ab = lax.dot_general(x, cb_ref[...], (((1,), (1,)), ((), ())),
                         preferred_element_type=jnp.float32)
    d = a2_ref[...][:, None] - 2.0 * ab + b2_ref[...][None, :]
    minval = jnp.min(d, axis=1, keepdims=True)
    iota = lax.broadcasted_iota(jnp.int32, (_BLK, _K), 1)
    idx = jnp.min(jnp.where(d == minval, iota, _K), axis=1)
    idx_ref[...] = idx
    bsum_ref[i] = jnp.sum(minval)


def _dist_argmin(x2, cb, a2, b2):
    return pl.pallas_call(
        _dist_argmin_kernel,
        grid=(_NBLK,),
        in_specs=[
            pl.BlockSpec((_BLK, _D), lambda i: (i, 0)),
            pl.BlockSpec((_K, _D), lambda i: (0, 0)),
            pl.BlockSpec((_BLK,), lambda i: (i,)),
            pl.BlockSpec((_K,), lambda i: (0,)),
        ],
        out_specs=[
            pl.BlockSpec((_BLK,), lambda i: (i,)),
            pl.BlockSpec(memory_space=pltpu.SMEM),
        ],
        out_shape=[
            jax.ShapeDtypeStruct((_NSUB,), jnp.int32),
            jax.ShapeDtypeStruct((_NBLK,), jnp.float32),
        ],
    )(x2, cb, a2, b2)


def _sc_gather_chunk(cbq, idx, row0, out_ref):
    mesh = plsc.VectorSubcoreMesh(core_axis_name="c", subcore_axis_name="s")

    @functools.partial(
        pl.kernel, mesh=mesh,
        scratch_types=[
            pltpu.VMEM((_BPW,), jnp.int32),
            pltpu.VMEM((2, _CH, _D), jnp.float32),
            pltpu.SemaphoreType.DMA,
            pltpu.SemaphoreType.DMA,
        ],
    )
    def k(table_hbm, idx_hbm, out_hbm, idx_v, rows_v, sem0, sem1):
        wid = lax.axis_index("s") * 2 + lax.axis_index("c")
        base = wid * _BPW
        sems = (sem0, sem1)
        pltpu.sync_copy(idx_hbm.at[pl.ds(base, _BPW)], idx_v)
        pltpu.async_copy(
            table_hbm.at[idx_v.at[pl.ds(0, _CH)]], rows_v.at[0], sems[0])
        for c in range(_NCH):
            nxt = (c + 1) % 2
            if c + 1 < _NCH:
                pltpu.async_copy(
                    table_hbm.at[idx_v.at[pl.ds((c + 1) * _CH, _CH)]],
                    rows_v.at[nxt], sems[nxt])
            pltpu.make_async_copy(
                table_hbm.at[idx_v.at[pl.ds(c * _CH, _CH)]],
                rows_v.at[c % 2], sems[c % 2]).wait()
            pltpu.sync_copy(rows_v.at[c % 2],
                            out_hbm.at[pl.ds(row0 + base + c * _CH, _CH)])

    k(cbq, idx, out_ref)


def _round_bf16_rne(v):
    bits = lax.bitcast_convert_type(v, jnp.int32)
    rb = bits + 0x7FFF + ((bits >> 16) & 1)
    rb = rb & jnp.int32(-65536)
    return lax.bitcast_convert_type(rb, jnp.float32)


def kernel(x, codebook):
    x2 = x.reshape(_N, _D)
    a2 = jnp.sum(jnp.square(x2), axis=-1)
    b2 = jnp.sum(jnp.square(codebook), axis=-1)
    cbq = _round_bf16_rne(codebook)
    idxs, bsums = [], []
    qref = jax.new_ref(jnp.zeros((_N, _D), jnp.float32))
    for s in range(_NSPLIT):
        sl = slice(s * _NSUB, (s + 1) * _NSUB)
        idx_s, bsum_s = _dist_argmin(x2[sl], codebook, a2[sl], b2)
        idxs.append(idx_s)
        bsums.append(bsum_s)
        _sc_gather_chunk(cbq, idx_s, s * _NSUB, qref)
    idx = jnp.concatenate(idxs)
    quantized = qref[...].reshape(_B, _T, _D)
    loss = (jnp.sum(jnp.stack(bsums)) * ((1.0 + _COMMIT) / (_N * _D)))
    return quantized, loss, idx.reshape(_B, _T)


# speedup vs baseline: 1.2361x; 1.2361x over previous
import functools

import jax
import jax.numpy as jnp
from jax import lax
from jax.experimental import pallas as pl
from jax.experimental.pallas import tpu as pltpu
from jax.experimental.pallas import tpu_sc as plsc

_B, _T, _D = 16, 1024, 256
_K = 1024
_N = _B * _T
_BLK = 512
_NBLK = _N // _BLK
_COMMIT = 0.25


def _sumsq_rows(v):
    v2 = v * v
    h = v2[:, :128] + v2[:, 128:]
    ht = h.T
    acc = ht[0:8, :]
    for t in range(1, 16):
        acc = acc + ht[8 * t:8 * t + 8, :]
    a = [acc[s:s + 1, :] for s in range(8)]
    return (((a[5] + a[1]) + (a[7] + a[3]))
            + ((a[6] + a[2]) + (a[0] + a[4])))


def _round_bf16_rne(v):
    bits = lax.bitcast_convert_type(v, jnp.int32)
    rb = bits + 0x7FFF + ((bits >> 16) & 1)
    rb = rb & jnp.int32(-65536)
    return lax.bitcast_convert_type(rb, jnp.float32)


def _dist_argmin_kernel(x_ref, cb_ref, idx_ref, bsum_ref, cbq_ref, b2_ref):
    i = pl.program_id(0)

    @pl.when(i == 0)
    def _():
        cb = cb_ref[...]
        b2_ref[...] = _sumsq_rows(cb)
        cbq_ref[...] = _round_bf16_rne(cb)

    x = x_ref[...]
    a2 = _sumsq_rows(x).T
    ab = lax.dot_general(x, cb_ref[...], (((1,), (1,)), ((), ())),
                         preferred_element_type=jnp.float32)
    d = a2 - 2.0 * ab + b2_ref[...]
    minval = jnp.min(d, axis=1, keepdims=True)
    iota = lax.broadcasted_iota(jnp.int32, (_BLK, _K), 1)
    idx = jnp.min(jnp.where(d == minval, iota, _K), axis=1)
    idx_ref[...] = idx
    bsum_ref[i] = jnp.sum(minval)


def _dist_argmin(x2, cb):
    return pl.pallas_call(
        _dist_argmin_kernel,
        grid=(_NBLK,),
        in_specs=[
            pl.BlockSpec((_BLK, _D), lambda i: (i, 0)),
            pl.BlockSpec((_K, _D), lambda i: (0, 0)),
        ],
        out_specs=[
            pl.BlockSpec((_BLK,), lambda i: (i,)),
            pl.BlockSpec(memory_space=pltpu.SMEM),
            pl.BlockSpec((_K, _D), lambda i: (0, 0)),
        ],
        out_shape=[
            jax.ShapeDtypeStruct((_N,), jnp.int32),
            jax.ShapeDtypeStruct((_NBLK,), jnp.float32),
            jax.ShapeDtypeStruct((_K, _D), jnp.float32),
        ],
        scratch_shapes=[pltpu.VMEM((1, _K), jnp.float32)],
    )(x2, cb)


_NW = 32
_BPW = _N // _NW
_CH = 128
_NCH = _BPW // _CH


def _sc_gather(cbq, idx):
    mesh = plsc.VectorSubcoreMesh(core_axis_name="c", subcore_axis_name="s")

    @functools.partial(
        pl.kernel, mesh=mesh,
        out_type=jax.ShapeDtypeStruct((_N, _D), jnp.float32),
        scratch_types=[
            pltpu.VMEM((_BPW,), jnp.int32),
            pltpu.VMEM((2, _CH, _D), jnp.float32),
            pltpu.SemaphoreType.DMA,
            pltpu.SemaphoreType.DMA,
        ],
    )
    def k(table_hbm, idx_hbm, out_hbm, idx_v, rows_v, sem0, sem1):
        wid = lax.axis_index("s") * 2 + lax.axis_index("c")
        base = wid * _BPW
        sems = (sem0, sem1)
        pltpu.sync_copy(idx_hbm.at[pl.ds(base, _BPW)], idx_v)
        pltpu.async_copy(
            table_hbm.at[idx_v.at[pl.ds(0, _CH)]], rows_v.at[0], sems[0])
        for c in range(_NCH):
            nxt = (c + 1) % 2
            if c + 1 < _NCH:
                pltpu.async_copy(
                    table_hbm.at[idx_v.at[pl.ds((c + 1) * _CH, _CH)]],
                    rows_v.at[nxt], sems[nxt])
            pltpu.make_async_copy(
                table_hbm.at[idx_v.at[pl.ds(c * _CH, _CH)]],
                rows_v.at[c % 2], sems[c % 2]).wait()
            pltpu.sync_copy(rows_v.at[c % 2],
                            out_hbm.at[pl.ds(base + c * _CH, _CH)])

    return k(cbq, idx)


def kernel(x, codebook):
    x2 = x.reshape(_N, _D)
    idx, bsums, cbq = _dist_argmin(x2, codebook)
    quantized = _sc_gather(cbq, idx).reshape(_B, _T, _D)
    loss = jnp.sum(bsums) * ((1.0 + _COMMIT) / (_N * _D))
    return quantized, loss, idx.reshape(_B, _T)
